# baseline (speedup 1.0000x reference)
import functools

import jax
import jax.numpy as jnp
from jax import lax
from jax.experimental import pallas as pl
from jax.experimental.pallas import tpu as pltpu
from jax.experimental.pallas import tpu_sc as plsc

VOCAB = 100000
BATCH = 16384
DIM = 300

NC = 2
NS = 16
NW = NC * NS
MAXK = (DIM + NW - 1) // NW
REM = DIM - (MAXK - 1) * NW
OUT_CHUNK = 4096
VEC = 16
UNROLL = 8

_mesh = plsc.VectorSubcoreMesh(core_axis_name="c", subcore_axis_name="s")


@functools.partial(
    pl.kernel,
    mesh=_mesh,
    out_type=jax.ShapeDtypeStruct((DIM, BATCH), jnp.float32),
    scratch_types=[
        pltpu.VMEM((BATCH,), jnp.int32),
        pltpu.VMEM((VOCAB,), jnp.float32),
        pltpu.VMEM((2, OUT_CHUNK), jnp.float32),
        pltpu.SemaphoreType.DMA,
        pltpu.SemaphoreType.DMA,
    ],
    compiler_params=pltpu.CompilerParams(needs_layout_passes=False),
)
def _gather_rows_kernel(tabT, poses_hbm, out_hbm, poses_v, row_v, obuf,
                        rsem, wsem):
    wid = lax.axis_index("s") * NC + lax.axis_index("c")
    pltpu.sync_copy(poses_hbm, poses_v)

    def do_row(row):
        pltpu.sync_copy(tabT.at[row], row_v)
        pend = []
        for c in range(BATCH // OUT_CHUNK):
            if c >= 2:
                pend[c - 2].wait()

            def body(i, _, c=c):
                for u in range(UNROLL):
                    off = (i * UNROLL + u) * VEC
                    idx = poses_v[pl.ds(c * OUT_CHUNK + off, VEC)]
                    obuf[c % 2, pl.ds(off, VEC)] = plsc.load_gather(
                        row_v, [idx])
                return _

            lax.fori_loop(0, OUT_CHUNK // (VEC * UNROLL), body, None)
            pend.append(pltpu.async_copy(
                obuf.at[c % 2],
                out_hbm.at[row, pl.ds(c * OUT_CHUNK, OUT_CHUNK)], wsem))
        pend[-2].wait()
        pend[-1].wait()

    for k in range(MAXK - 1):
        do_row(wid + k * NW)

    @pl.when(wid < REM)
    def _():
        do_row(wid + (MAXK - 1) * NW)


def kernel(pretrained, poses):
    outT = _gather_rows_kernel(pretrained.T, poses.astype(jnp.int32))
    return outT.T


# speedup vs baseline: 1.0164x; 1.0164x over previous
import functools

import jax
import jax.numpy as jnp
from jax import lax
from jax.experimental import pallas as pl
from jax.experimental.pallas import tpu as pltpu
from jax.experimental.pallas import tpu_sc as plsc

VOCAB = 100000
BATCH = 16384
DIM = 300

NC = 2
NS = 16
NW = NC * NS
MAXK = (DIM + NW - 1) // NW
REM = DIM - (MAXK - 1) * NW
OUT_CHUNK = 4096
VEC = 16
NCHUNK = BATCH // OUT_CHUNK

_mesh = plsc.VectorSubcoreMesh(core_axis_name="c", subcore_axis_name="s")


@functools.partial(
    pl.kernel,
    mesh=_mesh,
    out_type=jax.ShapeDtypeStruct((DIM, BATCH), jnp.float32),
    scratch_types=[
        pltpu.VMEM((BATCH,), jnp.int32),
        pltpu.VMEM((VOCAB,), jnp.float32),
        pltpu.VMEM((2, OUT_CHUNK), jnp.float32),
        pltpu.SemaphoreType.DMA,
        pltpu.SemaphoreType.DMA,
    ],
    compiler_params=pltpu.CompilerParams(needs_layout_passes=False),
)
def _gather_rows_kernel(tabT, poses_hbm, out_hbm, poses_v, row_v, obuf,
                        rsem, wsem):
    wid = lax.axis_index("s") * NC + lax.axis_index("c")
    pltpu.sync_copy(poses_hbm, poses_v)

    def do_row(row, carry, drain):
        load = pltpu.async_copy(tabT.at[row], row_v, rsem)
        for cp in carry:
            cp.wait()
        load.wait()
        pend = []
        for c in range(NCHUNK):
            if c >= 2:
                pend[c - 2].wait()

            def body(i, _, c=c):
                idx = poses_v[pl.ds(c * OUT_CHUNK + i * VEC, VEC)]
                obuf[c % 2, pl.ds(i * VEC, VEC)] = plsc.load_gather(
                    row_v, [idx])
                return _

            lax.fori_loop(0, OUT_CHUNK // VEC, body, None)
            pend.append(pltpu.async_copy(
                obuf.at[c % 2],
                out_hbm.at[row, pl.ds(c * OUT_CHUNK, OUT_CHUNK)], wsem))
        if drain:
            pend[-2].wait()
            pend[-1].wait()
            return []
        return pend[-2:]

    carry = []
    for k in range(MAXK - 1):
        carry = do_row(wid + k * NW, carry, drain=(k == MAXK - 2))

    @pl.when(wid < REM)
    def _():
        do_row(wid + (MAXK - 1) * NW, [], drain=True)


def kernel(pretrained, poses):
    outT = _gather_rows_kernel(pretrained.T, poses.astype(jnp.int32))
    return outT.T
